# Initial kernel scaffold; baseline (speedup 1.0000x reference)
import jax
import jax.numpy as jnp
from jax.experimental import pallas as pl


def kernel(news_ids, category_ids, news_table, category_table):
    raise NotImplementedError("write your pallas kernel here")


# speedup vs baseline: 1.4442x; 1.4442x over previous
import functools

import jax
import jax.numpy as jnp
from jax import lax
from jax.experimental import pallas as pl
from jax.experimental.pallas import tpu as pltpu
from jax.experimental.pallas import tpu_sc as plsc

NUM_NEWS = 1000000
NUM_CATEGORIES = 1000
NEWS_DIM = 64
CATEGORY_DIM = 16
BATCH = 4096
SEQ_LEN = 50
TOTAL = BATCH * SEQ_LEN
JOINT_DIM = NEWS_DIM + CATEGORY_DIM

NUM_CORES = 2
NUM_SUBCORES = 16
NW = NUM_CORES * NUM_SUBCORES
PER_W = TOTAL // NW
CHUNK = 128
N_CHUNK = PER_W // CHUNK


def _sc_body(nidx_hbm, cidx_hbm, news_hbm, cat_hbm, out_hbm,
             nidx_v, cidx_v, news_v, cat_v, sem):
    cid = lax.axis_index("c")
    sid = lax.axis_index("s")
    wid = sid * NUM_CORES + cid
    pltpu.sync_copy(nidx_hbm.at[wid], nidx_v)
    pltpu.sync_copy(cidx_hbm.at[wid], cidx_v)

    def chunk_body(c, carry):
        pltpu.async_copy(news_hbm.at[nidx_v.at[c]], news_v, sem).wait()
        pltpu.async_copy(cat_hbm.at[cidx_v.at[c]], cat_v, sem).wait()
        row0 = (wid * N_CHUNK + c) * CHUNK
        pltpu.sync_copy(news_v, out_hbm.at[pl.ds(row0, CHUNK), pl.ds(0, NEWS_DIM)])
        pltpu.sync_copy(cat_v, out_hbm.at[pl.ds(row0, CHUNK), pl.ds(NEWS_DIM, CATEGORY_DIM)])
        return carry

    lax.fori_loop(0, N_CHUNK, chunk_body, 0)


@jax.jit
def _joint_embed(news_idx, cat_idx, news_table, category_table):
    mesh = plsc.VectorSubcoreMesh(core_axis_name="c", subcore_axis_name="s")
    f = functools.partial(
        pl.kernel,
        mesh=mesh,
        out_type=jax.ShapeDtypeStruct((TOTAL, JOINT_DIM), jnp.float32),
        scratch_types=[
            pltpu.VMEM((N_CHUNK, CHUNK), jnp.int32),
            pltpu.VMEM((N_CHUNK, CHUNK), jnp.int32),
            pltpu.VMEM((CHUNK, NEWS_DIM), jnp.float32),
            pltpu.VMEM((CHUNK, CATEGORY_DIM), jnp.float32),
            pltpu.SemaphoreType.DMA,
        ],
        compiler_params=pltpu.CompilerParams(use_tc_tiling_on_sc=False),
    )(_sc_body)
    return f(news_idx, cat_idx, news_table, category_table)


def kernel(news_ids, category_ids, news_table, category_table):
    news_idx = news_ids.reshape(NW, N_CHUNK, CHUNK)
    cat_idx = category_ids.reshape(NW, N_CHUNK, CHUNK)
    out = _joint_embed(news_idx, cat_idx, news_table, category_table)
    return out.reshape(BATCH, SEQ_LEN, JOINT_DIM)


# speedup vs baseline: 1.5423x; 1.0680x over previous
import functools

import jax
import jax.numpy as jnp
from jax import lax
from jax.experimental import pallas as pl
from jax.experimental.pallas import tpu as pltpu
from jax.experimental.pallas import tpu_sc as plsc

NUM_NEWS = 1000000
NUM_CATEGORIES = 1000
NEWS_DIM = 64
CATEGORY_DIM = 16
BATCH = 4096
SEQ_LEN = 50
TOTAL = BATCH * SEQ_LEN
JOINT_DIM = NEWS_DIM + CATEGORY_DIM

NUM_CORES = 2
NUM_SUBCORES = 16
NW = NUM_CORES * NUM_SUBCORES
PER_W = TOTAL // NW
CHUNK = 128
N_CHUNK = PER_W // CHUNK
DEPTH = 8
LAG = 4


def _sc_body(nidx_hbm, cidx_hbm, news_hbm, cat_hbm, out_hbm,
             nidx_v, cidx_v, news_v, cat_v, gsem, wsem):
    cid = lax.axis_index("c")
    sid = lax.axis_index("s")
    wid = sid * NUM_CORES + cid
    pltpu.sync_copy(nidx_hbm.at[wid], nidx_v)
    pltpu.sync_copy(cidx_hbm.at[wid], cidx_v)
    base_row = wid * PER_W

    def gather_copies(c):
        slot = lax.rem(c, DEPTH)
        return (
            pltpu.make_async_copy(news_hbm.at[nidx_v.at[c]], news_v.at[slot],
                                  gsem.at[slot]),
            pltpu.make_async_copy(cat_hbm.at[cidx_v.at[c]], cat_v.at[slot],
                                  gsem.at[slot]),
        )

    def write_copies(c):
        slot = lax.rem(c, DEPTH)
        row0 = base_row + c * CHUNK
        return (
            pltpu.make_async_copy(
                news_v.at[slot],
                out_hbm.at[pl.ds(row0, CHUNK), pl.ds(0, NEWS_DIM)],
                wsem.at[slot]),
            pltpu.make_async_copy(
                cat_v.at[slot],
                out_hbm.at[pl.ds(row0, CHUNK), pl.ds(NEWS_DIM, CATEGORY_DIM)],
                wsem.at[slot]),
        )

    def body(c, carry):
        @pl.when(c < N_CHUNK)
        def _():
            @pl.when(c >= DEPTH)
            def _():
                for cp in write_copies(c - DEPTH):
                    cp.wait()
            for cp in gather_copies(c):
                cp.start()
        c2 = c - LAG
        @pl.when(c2 >= 0)
        def _():
            for cp in gather_copies(c2):
                cp.wait()
            for cp in write_copies(c2):
                cp.start()
        return carry

    lax.fori_loop(0, N_CHUNK + LAG, body, 0)

    def drain(i, carry):
        for cp in write_copies(N_CHUNK - DEPTH + i):
            cp.wait()
        return carry

    lax.fori_loop(0, DEPTH, drain, 0)


@jax.jit
def _joint_embed(news_idx, cat_idx, news_table, category_table):
    mesh = plsc.VectorSubcoreMesh(core_axis_name="c", subcore_axis_name="s")
    f = functools.partial(
        pl.kernel,
        mesh=mesh,
        out_type=jax.ShapeDtypeStruct((TOTAL, JOINT_DIM), jnp.float32),
        scratch_types=[
            pltpu.VMEM((N_CHUNK, CHUNK), jnp.int32),
            pltpu.VMEM((N_CHUNK, CHUNK), jnp.int32),
            pltpu.VMEM((DEPTH, CHUNK, NEWS_DIM), jnp.float32),
            pltpu.VMEM((DEPTH, CHUNK, CATEGORY_DIM), jnp.float32),
            pltpu.SemaphoreType.DMA((DEPTH,)),
            pltpu.SemaphoreType.DMA((DEPTH,)),
        ],
        compiler_params=pltpu.CompilerParams(use_tc_tiling_on_sc=False),
    )(_sc_body)
    return f(news_idx, cat_idx, news_table, category_table)


def kernel(news_ids, category_ids, news_table, category_table):
    news_idx = news_ids.reshape(NW, N_CHUNK, CHUNK)
    cat_idx = category_ids.reshape(NW, N_CHUNK, CHUNK)
    out = _joint_embed(news_idx, cat_idx, news_table, category_table)
    return out.reshape(BATCH, SEQ_LEN, JOINT_DIM)
